# Initial kernel scaffold; baseline (speedup 1.0000x reference)
import jax
import jax.numpy as jnp
from jax.experimental import pallas as pl


def kernel(x, params):
    raise NotImplementedError("write your pallas kernel here")


# speedup vs baseline: 2.4338x; 2.4338x over previous
import functools
import math

import jax
import jax.numpy as jnp
from jax import lax
from jax.experimental import pallas as pl

F32 = jnp.float32
_KNN = 16
_INF = float("inf")


def _dot(a, b):
    return lax.dot_general(a, b, (((1,), (0,)), ((), ())),
                           preferred_element_type=F32)


def _dotx(a, b):
    return lax.dot_general(a, b, (((1,), (0,)), ((), ())),
                           preferred_element_type=F32,
                           precision=lax.Precision.HIGHEST)


def _w_spec(shape):
    nd = len(shape)
    return pl.BlockSpec(shape, lambda *_: (0,) * nd)


def _b_spec(shape):
    nd = len(shape)
    return pl.BlockSpec((1,) + shape, lambda b, *_: (b,) + (0,) * nd)


def _stem_body(x_ref, w0_ref, b0_ref, w1_ref, b1_ref, o_ref):
    h = jnp.maximum(_dot(x_ref[0], w0_ref[...]) + b0_ref[...], 0.0)
    o_ref[0] = _dot(h, w1_ref[...]) + b1_ref[...]


def _stem(xp, w0p, b0, w1, b1):
    bsz, n, _ = xp.shape
    return pl.pallas_call(
        _stem_body,
        grid=(bsz,),
        in_specs=[_b_spec((n, 8)), _w_spec(w0p.shape), _w_spec(b0.shape),
                  _w_spec(w1.shape), _w_spec(b1.shape)],
        out_specs=_b_spec((n, 32)),
        out_shape=jax.ShapeDtypeStruct((bsz, n, 32), F32),
    )(xp, w0p, b0, w1, b1)


def _qkv_body(f_ref, w_ref, b_ref, wq_ref, wk_ref, wv_ref,
              q_ref, k_ref, v_ref):
    x = _dot(f_ref[0], w_ref[...]) + b_ref[...]
    q_ref[0] = _dot(x, wq_ref[...])
    k_ref[0] = _dot(x, wk_ref[...])
    v_ref[0] = _dot(x, wv_ref[...])


def _qkv(feats, w, b, wq, wk, wv):
    bsz, n, din = feats.shape
    dm = wq.shape[0]
    out = jax.ShapeDtypeStruct((bsz, n, dm), F32)
    return pl.pallas_call(
        _qkv_body,
        grid=(bsz,),
        in_specs=[_b_spec((n, din)), _w_spec(w.shape), _w_spec(b.shape),
                  _w_spec(wq.shape), _w_spec(wk.shape), _w_spec(wv.shape)],
        out_specs=[_b_spec((n, dm))] * 3,
        out_shape=[out, out, out],
    )(feats, w, b, wq, wk, wv)


def _attn_body(n, k_eff, inv_sqrt,
               xq_ref, xt_ref, xf_ref, q_ref, kall_ref, vall_ref, f_ref,
               d1_ref, d1b_ref, d2_ref, d2b_ref,
               g1_ref, g1b_ref, g2_ref, g2b_ref,
               fc2_ref, fc2b_ref, o_ref):
    xq = xq_ref[0]
    xt = xt_ref[0]
    a2 = jnp.sum(xt * xt, axis=0, keepdims=True)
    q2 = jnp.sum(xq * xq, axis=1, keepdims=True)
    dd = q2 + a2 - 2.0 * _dotx(xq, xt)
    iota = lax.broadcasted_iota(jnp.int32, dd.shape, 1)
    q = q_ref[0]
    kall = kall_ref[0]
    vall = vall_ref[0]
    xall = xf_ref[0]
    logits = []
    vals = []
    for _ in range(k_eff):
        m = jnp.min(dd, axis=1, keepdims=True)
        idx = jnp.min(jnp.where(dd == m, iota, n), axis=1, keepdims=True)
        sel = iota == idx
        oh = sel.astype(F32)
        dd = jnp.where(sel, _INF, dd)
        kk = _dotx(oh, kall)
        vv = _dotx(oh, vall)
        kx = _dotx(oh, xall)
        pos = xq - kx
        pe = _dot(jnp.maximum(_dot(pos, d1_ref[...]) + d1b_ref[...], 0.0),
                  d2_ref[...]) + d2b_ref[...]
        a = _dot(jnp.maximum(_dot(q - kk + pe, g1_ref[...]) + g1b_ref[...],
                             0.0), g2_ref[...]) + g2b_ref[...]
        logits.append(a * inv_sqrt)
        vals.append(vv + pe)
    mx = logits[0]
    for a in logits[1:]:
        mx = jnp.maximum(mx, a)
    es = [jnp.exp(a - mx) for a in logits]
    s = es[0]
    for e in es[1:]:
        s = s + e
    r = es[0] * vals[0]
    for e, u in zip(es[1:], vals[1:]):
        r = r + e * u
    res = r / s
    o_ref[0] = _dot(res, fc2_ref[...]) + fc2b_ref[...] + f_ref[0]


def _attn(xp, xt, q, kall, vall, feats, p, k_eff):
    bsz, n, din = feats.shape
    dm = q.shape[-1]
    tile = 128 if n >= 128 else n
    ntile = n // tile
    d1, d1b = p["d1_wp"], p["d1_b"]
    body = functools.partial(_attn_body, n, k_eff, 1.0 / math.sqrt(dm))
    tiled = lambda w: pl.BlockSpec((1, tile, w), lambda b, t: (b, t, 0))
    full = lambda s0, s1: pl.BlockSpec((1, s0, s1), lambda b, t: (b, 0, 0))
    return pl.pallas_call(
        body,
        grid=(bsz, ntile),
        in_specs=[tiled(8), full(8, n), full(n, 8), tiled(dm),
                  full(n, dm), full(n, dm), tiled(din),
                  _w_spec(d1.shape), _w_spec(d1b.shape),
                  _w_spec(p["d2_w"].shape), _w_spec(p["d2_b2"].shape),
                  _w_spec(p["g1_w"].shape), _w_spec(p["g1_b2"].shape),
                  _w_spec(p["g2_w"].shape), _w_spec(p["g2_b2"].shape),
                  _w_spec(p["fc2_w"].shape), _w_spec(p["fc2_b2"].shape)],
        out_specs=tiled(din),
        out_shape=jax.ShapeDtypeStruct((bsz, n, din), F32),
    )(xp, xt, xp, q, kall, vall, feats,
      d1, d1b, p["d2_w"], p["d2_b2"], p["g1_w"], p["g1_b2"],
      p["g2_w"], p["g2_b2"], p["fc2_w"], p["fc2_b2"])


def _tf_block(xp, xt, feats, p, k_eff):
    q, kall, vall = _qkv(feats, p["fc1_w"], p["fc1_b2"],
                         p["wq"], p["wk"], p["wv"])
    return _attn(xp, xt, q, kall, vall, feats, p, k_eff)


def _sa_body(n, npoint, k_eff,
             xt_ref, fcat_ref, w1_ref, w1a_ref, b1_ref, w2_ref, b2_ref,
             nxyz_ref, npts_ref, s_ref):
    xt = xt_ref[0]
    a2 = jnp.sum(xt * xt, axis=0, keepdims=True)
    iota1 = lax.broadcasted_iota(jnp.int32, (1, n), 1)

    def step(i, carry):
        dist, far = carry
        oh = (iota1 == far).astype(F32)
        s_ref[pl.ds(i, 1), :] = oh
        cent = lax.dot_general(xt, oh, (((1,), (1,)), ((), ())),
                               preferred_element_type=F32,
                               precision=lax.Precision.HIGHEST)
        diff = xt - cent
        d = jnp.sum(diff * diff, axis=0, keepdims=True)
        dist = jnp.minimum(dist, d)
        mxv = jnp.max(dist)
        far = jnp.min(jnp.where(dist == mxv, iota1, n)).astype(jnp.int32)
        return dist, far

    dist0 = jnp.full((1, n), 1e10, dtype=F32)
    lax.fori_loop(0, npoint, step, (dist0, jnp.int32(0)))

    smat = s_ref[...]
    fcat = fcat_ref[0]
    nx = _dotx(smat, fcat[:, :8])
    c2 = jnp.sum(nx * nx, axis=1, keepdims=True)
    dd = c2 + a2 - 2.0 * _dotx(nx, xt)
    iota = lax.broadcasted_iota(jnp.int32, dd.shape, 1)
    nxw1a = _dot(nx, w1a_ref[...])
    mp = None
    for _ in range(k_eff):
        m = jnp.min(dd, axis=1, keepdims=True)
        idx = jnp.min(jnp.where(dd == m, iota, n), axis=1, keepdims=True)
        sel = iota == idx
        oh = sel.astype(F32)
        dd = jnp.where(sel, _INF, dd)
        g = _dotx(oh, fcat)
        h = jnp.maximum(_dot(g, w1_ref[...]) - nxw1a + b1_ref[...], 0.0)
        h = jnp.maximum(_dot(h, w2_ref[...]) + b2_ref[...], 0.0)
        mp = h if mp is None else jnp.maximum(mp, h)
    nxyz_ref[0] = nx
    npts_ref[0] = mp


def _sa(xp, xt, points, p, npoint, k_eff):
    bsz, n, din = points.shape
    dout = p["w2"].shape[1]
    fcat = jnp.concatenate([xp, points], axis=-1)
    body = functools.partial(_sa_body, n, npoint, k_eff)
    from jax.experimental.pallas import tpu as pltpu
    nxyz, npts = pl.pallas_call(
        body,
        grid=(bsz,),
        in_specs=[_b_spec((8, n)), _b_spec((n, 8 + din)),
                  _w_spec(p["w1"].shape), _w_spec(p["w1a"].shape),
                  _w_spec(p["b1"].shape), _w_spec(p["w2"].shape),
                  _w_spec(p["b2"].shape)],
        out_specs=[_b_spec((npoint, 8)), _b_spec((npoint, dout))],
        out_shape=[jax.ShapeDtypeStruct((bsz, npoint, 8), F32),
                   jax.ShapeDtypeStruct((bsz, npoint, dout), F32)],
        scratch_shapes=[pltpu.VMEM((npoint, n), F32)],
    )(xt, fcat, p["w1"], p["w1a"], p["b1"], p["w2"], p["b2"])
    return nxyz, npts


def _row2(b):
    return b.reshape(1, -1)


def _pad_rows(w, rows):
    return jnp.concatenate(
        [w, jnp.zeros((rows - w.shape[0], w.shape[1]), F32)], axis=0)


def _prep_tf(p):
    q = dict(p)
    q["fc1_b2"] = _row2(p["fc1_b"])
    q["d1_wp"] = _pad_rows(p["d1_w"], 8)
    q["d1_b"] = _row2(p["d1_b"])
    q["d2_b2"] = _row2(p["d2_b"])
    q["g1_b2"] = _row2(p["g1_b"])
    q["g2_b2"] = _row2(p["g2_b"])
    q["fc2_b2"] = _row2(p["fc2_b"])
    return q


def _fold_bn(w, b, gamma, beta, mean, var):
    s = gamma / jnp.sqrt(var + 1e-5)
    return w * s[None, :], (b - mean) * s + beta


def _prep_sa(p):
    w1, b1 = _fold_bn(p["conv1_w"], p["conv1_b"], p["bn1_gamma"],
                      p["bn1_beta"], p["bn1_mean"], p["bn1_var"])
    w2, b2 = _fold_bn(p["conv2_w"], p["conv2_b"], p["bn2_gamma"],
                      p["bn2_beta"], p["bn2_mean"], p["bn2_var"])
    w1p = jnp.concatenate(
        [_pad_rows(w1[:3], 8), w1[3:]], axis=0)
    return {"w1": w1p, "w1a": w1p[:8], "b1": _row2(b1),
            "w2": w2, "b2": _row2(b2)}


def kernel(x, params):
    bsz, n, _ = x.shape
    xyz = x[..., :3]
    xp = jnp.concatenate(
        [xyz, jnp.zeros(xyz.shape[:-1] + (5,), F32)], axis=-1)
    xt = jnp.swapaxes(xp, 1, 2)

    h = _stem(xp, _pad_rows(params["fc1_0_w"], 8), _row2(params["fc1_0_b"]),
              params["fc1_1_w"], _row2(params["fc1_1_b"]))

    pts = _tf_block(xp, xt, h, _prep_tf(params["tf0"]), min(_KNN, n))
    feats = [(xyz, pts)]
    cur_xp, cur_xt, cur_n = xp, xt, n
    for i in range(4):
        npoint = cur_n // 4
        nxyz, pts = _sa(cur_xp, cur_xt, pts, _prep_sa(params["td%d" % i]),
                        npoint, min(_KNN, cur_n))
        cur_xp = nxyz
        cur_xt = jnp.swapaxes(nxyz, 1, 2)
        cur_n = npoint
        pts = _tf_block(cur_xp, cur_xt, pts, _prep_tf(params["tf%d" % (i + 1)]),
                        min(_KNN, cur_n))
        feats.append((nxyz[..., :3], pts))

    outs = [pts]
    for xyz_i, f_i in feats:
        outs.append(xyz_i)
        outs.append(f_i)
    return tuple(outs)


# speedup vs baseline: 3.3235x; 1.3656x over previous
import functools
import math

import jax
import jax.numpy as jnp
from jax import lax
from jax.experimental import pallas as pl

F32 = jnp.float32
_KNN = 16
_INF = float("inf")


def _dot(a, b):
    return lax.dot_general(a, b, (((1,), (0,)), ((), ())),
                           preferred_element_type=F32)


def _dotx(a, b):
    return lax.dot_general(a, b, (((1,), (0,)), ((), ())),
                           preferred_element_type=F32,
                           precision=lax.Precision.HIGHEST)


def _split3(x):
    hi = x.astype(jnp.bfloat16).astype(F32)
    r = x - hi
    mid = r.astype(jnp.bfloat16).astype(F32)
    return hi, mid, r - mid


def _gather3(oh, parts):
    hi, mid, lo = parts
    return _dot(oh, hi) + _dot(oh, mid) + _dot(oh, lo)


def _w_spec(shape):
    nd = len(shape)
    return pl.BlockSpec(shape, lambda *_: (0,) * nd)


def _b_spec(shape):
    nd = len(shape)
    return pl.BlockSpec((1,) + shape, lambda b, *_: (b,) + (0,) * nd)


def _stem_body(x_ref, w0_ref, b0_ref, w1_ref, b1_ref, o_ref):
    h = jnp.maximum(_dot(x_ref[0], w0_ref[...]) + b0_ref[...], 0.0)
    o_ref[0] = _dot(h, w1_ref[...]) + b1_ref[...]


def _stem(xp, w0p, b0, w1, b1):
    bsz, n, _ = xp.shape
    return pl.pallas_call(
        _stem_body,
        grid=(bsz,),
        in_specs=[_b_spec((n, 8)), _w_spec(w0p.shape), _w_spec(b0.shape),
                  _w_spec(w1.shape), _w_spec(b1.shape)],
        out_specs=_b_spec((n, 32)),
        out_shape=jax.ShapeDtypeStruct((bsz, n, 32), F32),
    )(xp, w0p, b0, w1, b1)


def _qkv_body(f_ref, w_ref, b_ref, wq_ref, wk_ref, wv_ref,
              q_ref, k_ref, v_ref):
    x = _dot(f_ref[0], w_ref[...]) + b_ref[...]
    q_ref[0] = _dot(x, wq_ref[...])
    k_ref[0] = _dot(x, wk_ref[...])
    v_ref[0] = _dot(x, wv_ref[...])


def _qkv(feats, w, b, wq, wk, wv):
    bsz, n, din = feats.shape
    dm = wq.shape[0]
    out = jax.ShapeDtypeStruct((bsz, n, dm), F32)
    return pl.pallas_call(
        _qkv_body,
        grid=(bsz,),
        in_specs=[_b_spec((n, din)), _w_spec(w.shape), _w_spec(b.shape),
                  _w_spec(wq.shape), _w_spec(wk.shape), _w_spec(wv.shape)],
        out_specs=[_b_spec((n, dm))] * 3,
        out_shape=[out, out, out],
    )(feats, w, b, wq, wk, wv)


def _attn_body(n, k_eff, inv_sqrt,
               xq_ref, xt_ref, xf_ref, q_ref, kall_ref, vall_ref, f_ref,
               d1_ref, d1b_ref, d2_ref, d2b_ref,
               g1_ref, g1b_ref, g2_ref, g2b_ref,
               fc2_ref, fc2b_ref, o_ref):
    xq = xq_ref[0]
    xt = xt_ref[0]
    a2 = jnp.sum(xt * xt, axis=0, keepdims=True)
    q2 = jnp.sum(xq * xq, axis=1, keepdims=True)
    dd = q2 + a2 - 2.0 * _dotx(xq, xt)
    iota = lax.broadcasted_iota(jnp.int32, dd.shape, 1)
    q = q_ref[0]
    kall = kall_ref[0]
    vall = vall_ref[0]
    k3 = _split3(kall)
    v3 = _split3(vall)
    xall = xf_ref[0]
    logits = []
    vals = []
    for _ in range(k_eff):
        m = jnp.min(dd, axis=1, keepdims=True)
        idx = jnp.min(jnp.where(dd == m, iota, n), axis=1, keepdims=True)
        sel = iota == idx
        oh = sel.astype(F32)
        dd = jnp.where(sel, _INF, dd)
        kk = _gather3(oh, k3)
        vv = _gather3(oh, v3)
        kx = _dotx(oh, xall)
        pos = xq - kx
        pe = _dot(jnp.maximum(_dot(pos, d1_ref[...]) + d1b_ref[...], 0.0),
                  d2_ref[...]) + d2b_ref[...]
        a = _dot(jnp.maximum(_dot(q - kk + pe, g1_ref[...]) + g1b_ref[...],
                             0.0), g2_ref[...]) + g2b_ref[...]
        logits.append(a * inv_sqrt)
        vals.append(vv + pe)
    mx = logits[0]
    for a in logits[1:]:
        mx = jnp.maximum(mx, a)
    es = [jnp.exp(a - mx) for a in logits]
    s = es[0]
    for e in es[1:]:
        s = s + e
    r = es[0] * vals[0]
    for e, u in zip(es[1:], vals[1:]):
        r = r + e * u
    res = r / s
    o_ref[0] = _dot(res, fc2_ref[...]) + fc2b_ref[...] + f_ref[0]


def _attn(xp, xt, q, kall, vall, feats, p, k_eff):
    bsz, n, din = feats.shape
    dm = q.shape[-1]
    tile = 128 if n >= 128 else n
    ntile = n // tile
    d1, d1b = p["d1_wp"], p["d1_b"]
    body = functools.partial(_attn_body, n, k_eff, 1.0 / math.sqrt(dm))
    tiled = lambda w: pl.BlockSpec((1, tile, w), lambda b, t: (b, t, 0))
    full = lambda s0, s1: pl.BlockSpec((1, s0, s1), lambda b, t: (b, 0, 0))
    return pl.pallas_call(
        body,
        grid=(bsz, ntile),
        in_specs=[tiled(8), full(8, n), full(n, 8), tiled(dm),
                  full(n, dm), full(n, dm), tiled(din),
                  _w_spec(d1.shape), _w_spec(d1b.shape),
                  _w_spec(p["d2_w"].shape), _w_spec(p["d2_b2"].shape),
                  _w_spec(p["g1_w"].shape), _w_spec(p["g1_b2"].shape),
                  _w_spec(p["g2_w"].shape), _w_spec(p["g2_b2"].shape),
                  _w_spec(p["fc2_w"].shape), _w_spec(p["fc2_b2"].shape)],
        out_specs=tiled(din),
        out_shape=jax.ShapeDtypeStruct((bsz, n, din), F32),
    )(xp, xt, xp, q, kall, vall, feats,
      d1, d1b, p["d2_w"], p["d2_b2"], p["g1_w"], p["g1_b2"],
      p["g2_w"], p["g2_b2"], p["fc2_w"], p["fc2_b2"])


def _tf_block(xp, xt, feats, p, k_eff):
    q, kall, vall = _qkv(feats, p["fc1_w"], p["fc1_b2"],
                         p["wq"], p["wk"], p["wv"])
    return _attn(xp, xt, q, kall, vall, feats, p, k_eff)


def _sa_body(n, npoint, k_eff,
             xt_ref, fcat_ref, w1_ref, w1a_ref, b1_ref, w2_ref, b2_ref,
             nxyz_ref, npts_ref, s_ref):
    xt = xt_ref[0]
    a2 = jnp.sum(xt * xt, axis=0, keepdims=True)
    iota1 = lax.broadcasted_iota(jnp.int32, (1, n), 1)

    def step(i, carry):
        dist, far = carry
        oh = (iota1 == far).astype(F32)
        s_ref[pl.ds(i, 1), :] = oh
        cent = lax.dot_general(xt, oh, (((1,), (1,)), ((), ())),
                               preferred_element_type=F32,
                               precision=lax.Precision.HIGHEST)
        diff = xt - cent
        d = jnp.sum(diff * diff, axis=0, keepdims=True)
        dist = jnp.minimum(dist, d)
        mxv = jnp.max(dist)
        far = jnp.min(jnp.where(dist == mxv, iota1, n)).astype(jnp.int32)
        return dist, far

    dist0 = jnp.full((1, n), 1e10, dtype=F32)
    lax.fori_loop(0, npoint, step, (dist0, jnp.int32(0)))

    smat = s_ref[...]
    fcat = fcat_ref[0]
    nx = _dotx(smat, fcat[:, :8])
    c2 = jnp.sum(nx * nx, axis=1, keepdims=True)
    dd = c2 + a2 - 2.0 * _dotx(nx, xt)
    iota = lax.broadcasted_iota(jnp.int32, dd.shape, 1)
    nxw1a = _dot(nx, w1a_ref[...])
    f3 = _split3(fcat)
    mp = None
    for _ in range(k_eff):
        m = jnp.min(dd, axis=1, keepdims=True)
        idx = jnp.min(jnp.where(dd == m, iota, n), axis=1, keepdims=True)
        sel = iota == idx
        oh = sel.astype(F32)
        dd = jnp.where(sel, _INF, dd)
        g = _gather3(oh, f3)
        h = jnp.maximum(_dot(g, w1_ref[...]) - nxw1a + b1_ref[...], 0.0)
        h = jnp.maximum(_dot(h, w2_ref[...]) + b2_ref[...], 0.0)
        mp = h if mp is None else jnp.maximum(mp, h)
    nxyz_ref[0] = nx
    npts_ref[0] = mp


def _sa(xp, xt, points, p, npoint, k_eff):
    bsz, n, din = points.shape
    dout = p["w2"].shape[1]
    fcat = jnp.concatenate([xp, points], axis=-1)
    body = functools.partial(_sa_body, n, npoint, k_eff)
    from jax.experimental.pallas import tpu as pltpu
    nxyz, npts = pl.pallas_call(
        body,
        grid=(bsz,),
        in_specs=[_b_spec((8, n)), _b_spec((n, 8 + din)),
                  _w_spec(p["w1"].shape), _w_spec(p["w1a"].shape),
                  _w_spec(p["b1"].shape), _w_spec(p["w2"].shape),
                  _w_spec(p["b2"].shape)],
        out_specs=[_b_spec((npoint, 8)), _b_spec((npoint, dout))],
        out_shape=[jax.ShapeDtypeStruct((bsz, npoint, 8), F32),
                   jax.ShapeDtypeStruct((bsz, npoint, dout), F32)],
        scratch_shapes=[pltpu.VMEM((npoint, n), F32)],
    )(xt, fcat, p["w1"], p["w1a"], p["b1"], p["w2"], p["b2"])
    return nxyz, npts


def _row2(b):
    return b.reshape(1, -1)


def _pad_rows(w, rows):
    return jnp.concatenate(
        [w, jnp.zeros((rows - w.shape[0], w.shape[1]), F32)], axis=0)


def _prep_tf(p):
    q = dict(p)
    q["fc1_b2"] = _row2(p["fc1_b"])
    q["d1_wp"] = _pad_rows(p["d1_w"], 8)
    q["d1_b"] = _row2(p["d1_b"])
    q["d2_b2"] = _row2(p["d2_b"])
    q["g1_b2"] = _row2(p["g1_b"])
    q["g2_b2"] = _row2(p["g2_b"])
    q["fc2_b2"] = _row2(p["fc2_b"])
    return q


def _fold_bn(w, b, gamma, beta, mean, var):
    s = gamma / jnp.sqrt(var + 1e-5)
    return w * s[None, :], (b - mean) * s + beta


def _prep_sa(p):
    w1, b1 = _fold_bn(p["conv1_w"], p["conv1_b"], p["bn1_gamma"],
                      p["bn1_beta"], p["bn1_mean"], p["bn1_var"])
    w2, b2 = _fold_bn(p["conv2_w"], p["conv2_b"], p["bn2_gamma"],
                      p["bn2_beta"], p["bn2_mean"], p["bn2_var"])
    w1p = jnp.concatenate(
        [_pad_rows(w1[:3], 8), w1[3:]], axis=0)
    return {"w1": w1p, "w1a": w1p[:8], "b1": _row2(b1),
            "w2": w2, "b2": _row2(b2)}


def kernel(x, params):
    bsz, n, _ = x.shape
    xyz = x[..., :3]
    xp = jnp.concatenate(
        [xyz, jnp.zeros(xyz.shape[:-1] + (5,), F32)], axis=-1)
    xt = jnp.swapaxes(xp, 1, 2)

    h = _stem(xp, _pad_rows(params["fc1_0_w"], 8), _row2(params["fc1_0_b"]),
              params["fc1_1_w"], _row2(params["fc1_1_b"]))

    pts = _tf_block(xp, xt, h, _prep_tf(params["tf0"]), min(_KNN, n))
    feats = [(xyz, pts)]
    cur_xp, cur_xt, cur_n = xp, xt, n
    for i in range(4):
        npoint = cur_n // 4
        nxyz, pts = _sa(cur_xp, cur_xt, pts, _prep_sa(params["td%d" % i]),
                        npoint, min(_KNN, cur_n))
        cur_xp = nxyz
        cur_xt = jnp.swapaxes(nxyz, 1, 2)
        cur_n = npoint
        pts = _tf_block(cur_xp, cur_xt, pts, _prep_tf(params["tf%d" % (i + 1)]),
                        min(_KNN, cur_n))
        feats.append((nxyz[..., :3], pts))

    outs = [pts]
    for xyz_i, f_i in feats:
        outs.append(xyz_i)
        outs.append(f_i)
    return tuple(outs)
